# baseline (speedup 1.0000x reference)
import functools

import jax
import jax.numpy as jnp
from jax import lax
from jax.experimental import pallas as pl
from jax.experimental.pallas import tpu as pltpu
from jax.experimental.pallas import tpu_sc as plsc

N = 10000
F = 10000
H = 64
WCAT = 3 * H
NUM_CLASSES = 32

K = 128
GROUP = K * 16 * 6
ROWS_PER_TILE = 624


def _pad_to(x, total, axis):
    pad = total - x.shape[axis]
    cfg = [(0, 0)] * x.ndim
    cfg[axis] = (0, pad)
    return jnp.pad(x, cfg)


def _make_sc_spmm(nnz_pad, half):
    chunks = nnz_pad // K
    cpt = chunks // 16
    assert chunks % (16 * 3) == 0
    mesh = plsc.VectorSubcoreMesh(core_axis_name="c", subcore_axis_name="s")

    @functools.partial(
        pl.kernel,
        mesh=mesh,
        compiler_params=pltpu.CompilerParams(use_tc_tiling_on_sc=False),
        out_type=jax.ShapeDtypeStruct((2, N, half), jnp.float32),
        scratch_types=[
            [pltpu.VMEM((2, K), jnp.int32) for _ in range(3)],
            [pltpu.VMEM((K, 16), jnp.float32) for _ in range(3)],
            [pltpu.VMEM((K, half), jnp.float32) for _ in range(3)],
            pltpu.VMEM_SHARED((N, half), jnp.float32),
            [pltpu.SemaphoreType.DMA for _ in range(3)],
            [pltpu.SemaphoreType.DMA for _ in range(3)],
            [pltpu.SemaphoreType.DMA for _ in range(3)],
        ],
    )
    def spmm(pidx_hbm, valx_hbm, taba_hbm, tabb_hbm, z_hbm, out_hbm,
             idx, val, rows, acc, isem, gsem, ssem):
        cid = lax.axis_index("c")
        sid = lax.axis_index("s")

        r0 = sid * ROWS_PER_TILE
        tail0 = 16 * ROWS_PER_TILE
        tail_n = N - tail0
        pltpu.sync_copy(z_hbm.at[pl.ds(r0, ROWS_PER_TILE), :],
                        acc.at[pl.ds(r0, ROWS_PER_TILE), :])

        @pl.when(sid == 15)
        def _zero_tail():
            pltpu.sync_copy(z_hbm.at[pl.ds(tail0, tail_n), :],
                            acc.at[pl.ds(tail0, tail_n), :])

        plsc.subcore_barrier()
        c0 = sid * cpt

        def run(tab_hbm):
            def scale(rows_v, valx_v):
                @plsc.parallel_loop(0, K, unroll=8)
                def _scale(j):
                    v = valx_v[j, :]
                    for g in range(half // 16):
                        sl = pl.ds(g * 16, 16)
                        rows_v[j, sl] = rows_v[j, sl] * v

            def prefetch_idx(j, b):
                pltpu.async_copy(pidx_hbm.at[c0 + j], idx[b], isem[b])
                pltpu.async_copy(valx_hbm.at[c0 + j], val[b], isem[b])

            def launch_gather(j, b):
                pltpu.make_async_copy(pidx_hbm.at[c0 + j], idx[b],
                                      isem[b]).wait()
                pltpu.make_async_copy(valx_hbm.at[c0 + j], val[b],
                                      isem[b]).wait()
                pltpu.async_copy(tab_hbm.at[idx[b].at[0]], rows[b], gsem[b])

            def step(j, b0, b1, b2):
                @pl.when(j > 0)
                def _drain_prev_scatter():
                    pltpu.make_async_copy(rows[b2], acc.at[idx[b2].at[1]],
                                          ssem[b2]).wait()

                @pl.when(j + 1 < cpt)
                def _launch_next():
                    launch_gather(j + 1, b1)

                @pl.when(j + 2 < cpt)
                def _prefetch_next2():
                    prefetch_idx(j + 2, b2)

                pltpu.make_async_copy(tab_hbm.at[idx[b0].at[0]], rows[b0],
                                      gsem[b0]).wait()
                scale(rows[b0], val[b0])
                pltpu.async_copy(rows[b0], acc.at[idx[b0].at[1]], ssem[b0],
                                 add=True)

            prefetch_idx(0, 0)
            prefetch_idx(1, 1)
            launch_gather(0, 0)

            @pl.loop(0, cpt, step=3)
            def _triple(i):
                step(i, 0, 1, 2)
                step(i + 1, 1, 2, 0)
                step(i + 2, 2, 0, 1)

            b_last = (cpt - 1) % 3
            pltpu.make_async_copy(rows[b_last], acc.at[idx[b_last].at[1]],
                                  ssem[b_last]).wait()

        @pl.when(cid == 0)
        def _run_a():
            run(taba_hbm)

        @pl.when(cid == 1)
        def _run_b():
            run(tabb_hbm)

        plsc.subcore_barrier()
        pltpu.sync_copy(acc.at[pl.ds(r0, ROWS_PER_TILE), :],
                        out_hbm.at[cid, pl.ds(r0, ROWS_PER_TILE), :])

        @pl.when(sid == 15)
        def _write_tail():
            pltpu.sync_copy(acc.at[pl.ds(tail0, tail_n), :],
                            out_hbm.at[cid, pl.ds(tail0, tail_n), :])

    return spmm


ROW_BLK = 1000


def _tc_combine1_body(p_ref, b_ref, x64_ref, ya_ref, yb_ref):
    x = jnp.concatenate([p_ref[0], p_ref[1]], axis=1)
    x = jnp.maximum(x + b_ref[0][None, :], 0.0)
    x64_ref[...] = x[:, :H]
    ya_ref[...] = x[:, H:2 * H]
    yb_ref[...] = x[:, 2 * H:]


def _tc_combine1(p, bcat):
    grid = N // ROW_BLK
    return pl.pallas_call(
        _tc_combine1_body,
        grid=(grid,),
        in_specs=[
            pl.BlockSpec((2, ROW_BLK, WCAT // 2), lambda i: (0, i, 0)),
            pl.BlockSpec((1, WCAT), lambda i: (0, 0)),
        ],
        out_specs=[
            pl.BlockSpec((ROW_BLK, H), lambda i: (i, 0)),
            pl.BlockSpec((ROW_BLK, H), lambda i: (i, 0)),
            pl.BlockSpec((ROW_BLK, H), lambda i: (i, 0)),
        ],
        out_shape=[
            jax.ShapeDtypeStruct((N, H), jnp.float32),
            jax.ShapeDtypeStruct((N, H), jnp.float32),
            jax.ShapeDtypeStruct((N, H), jnp.float32),
        ],
    )(p, bcat)


def _tc_final_body(x_ref, t_ref, r_ref, wfc_ref, bfc_ref, out_ref):
    a2 = jnp.concatenate([r_ref[0], r_ref[1]], axis=1)
    w = wfc_ref[...]
    logits = jnp.dot(x_ref[...], w[:H], preferred_element_type=jnp.float32)
    logits += jnp.dot(t_ref[...], w[H:2 * H], preferred_element_type=jnp.float32)
    logits += jnp.dot(a2, w[2 * H:], preferred_element_type=jnp.float32)
    logits += bfc_ref[0][None, :]
    m = jnp.max(logits, axis=1, keepdims=True)
    z = logits - m
    lse = jnp.log(jnp.sum(jnp.exp(z), axis=1, keepdims=True))
    out_ref[...] = z - lse


def _tc_final(x64, t64, r, w_fc, b_fc):
    grid = N // ROW_BLK
    return pl.pallas_call(
        _tc_final_body,
        grid=(grid,),
        in_specs=[
            pl.BlockSpec((ROW_BLK, H), lambda i: (i, 0)),
            pl.BlockSpec((ROW_BLK, H), lambda i: (i, 0)),
            pl.BlockSpec((2, ROW_BLK, H // 2), lambda i: (0, i, 0)),
            pl.BlockSpec((WCAT, NUM_CLASSES), lambda i: (0, 0)),
            pl.BlockSpec((1, NUM_CLASSES), lambda i: (0, 0)),
        ],
        out_specs=pl.BlockSpec((ROW_BLK, NUM_CLASSES), lambda i: (i, 0)),
        out_shape=jax.ShapeDtypeStruct((N, NUM_CLASSES), jnp.float32),
    )(x64, t64, r, w_fc, b_fc.reshape(1, NUM_CLASSES))


def _ceil_to(x, m):
    return ((x + m - 1) // m) * m


def kernel(adj_indices, adj_values, feat_indices, feat_values,
           W1, b1, W2, b2, W3, b3, W_fc, b_fc):
    fpad = _ceil_to(feat_indices.shape[1], GROUP)
    apad = _ceil_to(adj_indices.shape[1], GROUP)

    w_a = jnp.concatenate([W1, W2[:, :H // 2]], axis=1)
    w_b = jnp.concatenate([W2[:, H // 2:], W3], axis=1)
    bcat = jnp.concatenate([b1, b2, b3], axis=1)

    def pack(indices, values, pad):
        col = _pad_to(indices[1], pad, 0).reshape(pad // K, 1, K)
        row = _pad_to(indices[0], pad, 0).reshape(pad // K, 1, K)
        pidx = jnp.concatenate([col, row], axis=1)
        valx = jnp.broadcast_to(
            _pad_to(values, pad, 0).reshape(pad // K, K)[:, :, None],
            (pad // K, K, 16))
        return pidx, valx

    f_pidx, f_valx = pack(feat_indices, feat_values, fpad)
    a_pidx, a_valx = pack(adj_indices, adj_values, apad)

    z96 = jnp.zeros((N, WCAT // 2), jnp.float32)
    z64 = jnp.zeros((N, H), jnp.float32)
    z32 = jnp.zeros((N, H // 2), jnp.float32)

    p = _make_sc_spmm(fpad, WCAT // 2)(f_pidx, f_valx, w_a, w_b, z96)
    x64, y_a, y_b = _tc_combine1(p, bcat)

    q = _make_sc_spmm(apad, H)(a_pidx, a_valx, y_a, y_b, z64)
    t64a, t64b = q[0], q[1]

    r = _make_sc_spmm(apad, H // 2)(
        a_pidx, a_valx, t64b[:, :H // 2], t64b[:, H // 2:], z32)

    return _tc_final(x64, t64a, r, W_fc, b_fc)


# speedup vs baseline: 1.2999x; 1.2999x over previous
import functools

import jax
import jax.numpy as jnp
from jax import lax
from jax.experimental import pallas as pl
from jax.experimental.pallas import tpu as pltpu
from jax.experimental.pallas import tpu_sc as plsc

N = 10000
F = 10000
H = 64
WCAT = 3 * H
NUM_CLASSES = 32

K = 128
GROUP = K * 32
ROWS_PER_TILE = 624

_GDN = lax.GatherDimensionNumbers(
    offset_dims=(), collapsed_slice_dims=(0,), start_index_map=(0,))


def _lane_bcast(v16, lane):
    idx = jnp.full((16, 1), lane, jnp.int32)
    return lax.gather(v16, idx, dimension_numbers=_GDN, slice_sizes=(1,),
                      mode=lax.GatherScatterMode.PROMISE_IN_BOUNDS)


def _pad_to(x, total, axis):
    pad = total - x.shape[axis]
    cfg = [(0, 0)] * x.ndim
    cfg[axis] = (0, pad)
    return jnp.pad(x, cfg)


def _make_sc_spmm(nnz_pad, half):
    chunks = nnz_pad // K
    cpt = chunks // 16
    assert chunks % 32 == 0
    mesh = plsc.VectorSubcoreMesh(core_axis_name="c", subcore_axis_name="s")

    @functools.partial(
        pl.kernel,
        mesh=mesh,
        compiler_params=pltpu.CompilerParams(use_tc_tiling_on_sc=False,
                                             needs_layout_passes=False),
        out_type=[jax.ShapeDtypeStruct((N, half), jnp.float32),
                  jax.ShapeDtypeStruct((N, half), jnp.float32)],
        scratch_types=[
            [pltpu.VMEM((3, K), jnp.int32) for _ in range(2)],
            [pltpu.VMEM((K, half), jnp.float32) for _ in range(2)],
            pltpu.VMEM_SHARED((N, half), jnp.float32),
            [pltpu.SemaphoreType.DMA for _ in range(2)],
            [pltpu.SemaphoreType.DMA for _ in range(2)],
        ],
    )
    def spmm(pidx_hbm, taba_hbm, tabb_hbm, z_hbm, outa_hbm, outb_hbm,
             idx, rows, acc, gsem, ssem):
        cid = lax.axis_index("c")
        sid = lax.axis_index("s")

        r0 = sid * ROWS_PER_TILE
        tail0 = 16 * ROWS_PER_TILE
        tail_n = N - tail0
        pltpu.sync_copy(z_hbm.at[pl.ds(r0, ROWS_PER_TILE), :],
                        acc.at[pl.ds(r0, ROWS_PER_TILE), :])

        @pl.when(sid == 15)
        def _zero_tail():
            pltpu.sync_copy(z_hbm.at[pl.ds(tail0, tail_n), :],
                            acc.at[pl.ds(tail0, tail_n), :])

        plsc.subcore_barrier()
        c0 = sid * cpt

        def run(tab_hbm):
            def scale(b):
                idx_v, rows_v = idx[b], rows[b]

                @plsc.parallel_loop(0, K // 16, unroll=2)
                def _scale(jg):
                    v16 = plsc.bitcast(idx_v[2, pl.ds(jg * 16, 16)],
                                       jnp.float32)
                    for l in range(16):
                        j = jg * 16 + l
                        bv = _lane_bcast(v16, l)
                        for g in range(half // 16):
                            sl = pl.ds(g * 16, 16)
                            rows_v[j, sl] = rows_v[j, sl] * bv

            def step(j, bA, bB):
                @pl.when(j > 0)
                def _drain_prev_scatter():
                    pltpu.make_async_copy(rows[bB], acc.at[idx[bB].at[1]],
                                          ssem[bB]).wait()

                @pl.when(j + 1 < cpt)
                def _prefetch_next():
                    pltpu.async_copy(pidx_hbm.at[c0 + j + 1], idx[bB],
                                     gsem[bB])

                pltpu.make_async_copy(tab_hbm.at[idx[bA].at[0]], rows[bA],
                                      gsem[bA]).wait()
                scale(bA)

                @pl.when(j + 1 < cpt)
                def _launch_next_gather():
                    pltpu.make_async_copy(pidx_hbm.at[c0 + j + 1], idx[bB],
                                          gsem[bB]).wait()
                    pltpu.async_copy(tab_hbm.at[idx[bB].at[0]], rows[bB],
                                     gsem[bB])

                pltpu.async_copy(rows[bA], acc.at[idx[bA].at[1]], ssem[bA],
                                 add=True)

            pltpu.async_copy(pidx_hbm.at[c0], idx[0], gsem[0])
            pltpu.make_async_copy(pidx_hbm.at[c0], idx[0], gsem[0]).wait()
            pltpu.async_copy(tab_hbm.at[idx[0].at[0]], rows[0], gsem[0])

            @pl.loop(0, cpt, step=2)
            def _pair(i):
                step(i, 0, 1)
                step(i + 1, 1, 0)

            pltpu.make_async_copy(rows[1], acc.at[idx[1].at[1]],
                                  ssem[1]).wait()

        @pl.when(cid == 0)
        def _run_a():
            run(taba_hbm)

        @pl.when(cid == 1)
        def _run_b():
            run(tabb_hbm)

        plsc.subcore_barrier()

        def writeout(out_hbm):
            pltpu.sync_copy(acc.at[pl.ds(r0, ROWS_PER_TILE), :],
                            out_hbm.at[pl.ds(r0, ROWS_PER_TILE), :])

            @pl.when(sid == 15)
            def _write_tail():
                pltpu.sync_copy(acc.at[pl.ds(tail0, tail_n), :],
                                out_hbm.at[pl.ds(tail0, tail_n), :])

        @pl.when(cid == 0)
        def _write_a():
            writeout(outa_hbm)

        @pl.when(cid == 1)
        def _write_b():
            writeout(outb_hbm)

    return spmm


ROW_BLK = 1000


def _tc_combine1_body(pa_ref, pb_ref, b_ref, x64_ref, ya_ref, yb_ref):
    x = jnp.concatenate([pa_ref[...], pb_ref[...]], axis=1)
    x = jnp.maximum(x + b_ref[0][None, :], 0.0)
    x64_ref[...] = x[:, :H]
    ya_ref[...] = x[:, H:2 * H]
    yb_ref[...] = x[:, 2 * H:]


def _tc_combine1(pa, pb, bcat):
    grid = N // ROW_BLK
    return pl.pallas_call(
        _tc_combine1_body,
        grid=(grid,),
        in_specs=[
            pl.BlockSpec((ROW_BLK, WCAT // 2), lambda i: (i, 0)),
            pl.BlockSpec((ROW_BLK, WCAT // 2), lambda i: (i, 0)),
            pl.BlockSpec((1, WCAT), lambda i: (0, 0)),
        ],
        out_specs=[
            pl.BlockSpec((ROW_BLK, H), lambda i: (i, 0)),
            pl.BlockSpec((ROW_BLK, H), lambda i: (i, 0)),
            pl.BlockSpec((ROW_BLK, H), lambda i: (i, 0)),
        ],
        out_shape=[
            jax.ShapeDtypeStruct((N, H), jnp.float32),
            jax.ShapeDtypeStruct((N, H), jnp.float32),
            jax.ShapeDtypeStruct((N, H), jnp.float32),
        ],
    )(pa, pb, bcat)


def _tc_final_body(x_ref, t_ref, ra_ref, rb_ref, wfc_ref, bfc_ref, out_ref):
    a2 = jnp.concatenate([ra_ref[...], rb_ref[...]], axis=1)
    w = wfc_ref[...]
    logits = jnp.dot(x_ref[...], w[:H], preferred_element_type=jnp.float32)
    logits += jnp.dot(t_ref[...], w[H:2 * H], preferred_element_type=jnp.float32)
    logits += jnp.dot(a2, w[2 * H:], preferred_element_type=jnp.float32)
    logits += bfc_ref[0][None, :]
    m = jnp.max(logits, axis=1, keepdims=True)
    z = logits - m
    lse = jnp.log(jnp.sum(jnp.exp(z), axis=1, keepdims=True))
    out_ref[...] = z - lse


def _tc_final(x64, t64, ra, rb, w_fc, b_fc):
    grid = N // ROW_BLK
    return pl.pallas_call(
        _tc_final_body,
        grid=(grid,),
        in_specs=[
            pl.BlockSpec((ROW_BLK, H), lambda i: (i, 0)),
            pl.BlockSpec((ROW_BLK, H), lambda i: (i, 0)),
            pl.BlockSpec((ROW_BLK, H // 2), lambda i: (i, 0)),
            pl.BlockSpec((ROW_BLK, H // 2), lambda i: (i, 0)),
            pl.BlockSpec((WCAT, NUM_CLASSES), lambda i: (0, 0)),
            pl.BlockSpec((1, NUM_CLASSES), lambda i: (0, 0)),
        ],
        out_specs=pl.BlockSpec((ROW_BLK, NUM_CLASSES), lambda i: (i, 0)),
        out_shape=jax.ShapeDtypeStruct((N, NUM_CLASSES), jnp.float32),
    )(x64, t64, ra, rb, w_fc, b_fc.reshape(1, NUM_CLASSES))


def _ceil_to(x, m):
    return ((x + m - 1) // m) * m


def kernel(adj_indices, adj_values, feat_indices, feat_values,
           W1, b1, W2, b2, W3, b3, W_fc, b_fc):
    fpad = _ceil_to(feat_indices.shape[1], GROUP)
    apad = _ceil_to(adj_indices.shape[1], GROUP)

    w_a = jnp.concatenate([W1, W2[:, :H // 2]], axis=1)
    w_b = jnp.concatenate([W2[:, H // 2:], W3], axis=1)
    bcat = jnp.concatenate([b1, b2, b3], axis=1)

    def pack(indices, values, pad):
        col = _pad_to(indices[1], pad, 0).reshape(pad // K, 1, K)
        row = _pad_to(indices[0], pad, 0).reshape(pad // K, 1, K)
        vbits = lax.bitcast_convert_type(
            _pad_to(values, pad, 0), jnp.int32).reshape(pad // K, 1, K)
        return jnp.concatenate([col, row, vbits], axis=1)

    f_pidx = pack(feat_indices, feat_values, fpad)
    a_pidx = pack(adj_indices, adj_values, apad)

    z96 = jnp.zeros((N, WCAT // 2), jnp.float32)
    z64 = jnp.zeros((N, H), jnp.float32)
    z32 = jnp.zeros((N, H // 2), jnp.float32)

    pa, pb = _make_sc_spmm(fpad, WCAT // 2)(f_pidx, w_a, w_b, z96)
    x64, y_a, y_b = _tc_combine1(pa, pb, bcat)

    t64a, t64b = _make_sc_spmm(apad, H)(a_pidx, y_a, y_b, z64)

    ra, rb = _make_sc_spmm(apad, H // 2)(
        a_pidx, t64b[:, :H // 2], t64b[:, H // 2:], z32)

    return _tc_final(x64, t64a, ra, rb, W_fc, b_fc)
